# baseline (speedup 1.0000x reference)
import math

import jax
import jax.numpy as jnp
from jax import lax
from jax.experimental import pallas as pl
from jax.experimental.pallas import tpu as pltpu
from jax.experimental.pallas import tpu_sc as plsc

_NC = 2
_NS = 16
_NW = _NC * _NS

_BATCH = 16384
_SEQ = 50
_EMB = 32
_TOTAL = _BATCH * _SEQ
_SCALE = math.sqrt(float(_EMB))

_LANE = 128
_TROW = 4
_CH = _TROW * _LANE
_ROWS_PER_W = _TOTAL // _NW // _LANE
_NCHUNK = _ROWS_PER_W // _TROW
_OUT_ROWS_CH = _CH * _EMB // _LANE


def _emb_body(tok_hbm, tab_hbm, out_hbm, tok_v, gidx_v, rows_v, out_v, sem):
    wid = lax.axis_index("s") * _NC + lax.axis_index("c")
    tok_base = wid * _ROWS_PER_W
    out_base = wid * (_ROWS_PER_W * _EMB)
    iota = lax.iota(jnp.int32, 16)

    def chunk(c, carry):
        trow = tok_base + c * _TROW
        pltpu.sync_copy(tok_hbm.at[pl.ds(trow, _TROW)], tok_v)
        for j in range(_TROW):
            def gidx_row(k, carry2, j=j):
                gidx_v[j, pl.ds(k * 16, 16)] = tok_v[j, pl.ds(k * 16, 16)] >> 2
                return carry2
            lax.fori_loop(0, _LANE // 16, gidx_row, 0)
        cps = [
            pltpu.async_copy(
                tab_hbm.at[gidx_v.at[j]],
                rows_v.at[pl.ds(j * _LANE, _LANE)],
                sem,
            )
            for j in range(_TROW)
        ]
        for cp in cps:
            cp.wait()

        for j in range(_TROW):
            def grp(k, carry2, j=j):
                g = j * (_LANE // 16) + k
                tokv = tok_v[j, pl.ds(k * 16, 16)]
                colb = (tokv & 3) << 5
                rowv = g * 16 + iota
                flat0 = rowv << 5

                def col(c2, state):
                    colv, flatv = state
                    val = plsc.load_gather(rows_v, [rowv, colv])
                    plsc.store_scatter(
                        out_v, [flatv >> 7, flatv & 127], val * _SCALE
                    )
                    return (colv + 1, flatv + 1)

                lax.fori_loop(0, _EMB, col, (colb, flat0))
                return carry2
            lax.fori_loop(0, _LANE // 16, grp, 0)

        pltpu.sync_copy(out_v, out_hbm.at[pl.ds(out_base + c * _OUT_ROWS_CH, _OUT_ROWS_CH)])
        return carry

    lax.fori_loop(0, _NCHUNK, chunk, 0)


_mesh = plsc.VectorSubcoreMesh(
    core_axis_name="c", subcore_axis_name="s", num_cores=_NC, num_subcores=_NS
)

_emb_call = pl.kernel(
    _emb_body,
    out_type=jax.ShapeDtypeStruct((_TOTAL * _EMB // _LANE, _LANE), jnp.float32),
    mesh=_mesh,
    scratch_types=[
        pltpu.VMEM((_TROW, _LANE), jnp.int32),
        pltpu.VMEM((_TROW, _LANE), jnp.int32),
        pltpu.VMEM((_CH, _LANE), jnp.float32),
        pltpu.VMEM((_OUT_ROWS_CH, _LANE), jnp.float32),
        pltpu.SemaphoreType.DMA,
    ],
    compiler_params=pltpu.CompilerParams(
        use_tc_tiling_on_sc=False, needs_layout_passes=False
    ),
)


@jax.jit
def kernel(tokens, embedding):
    tok = tokens.reshape(_TOTAL // _LANE, _LANE)
    tab = embedding.reshape(1000000 * _EMB // _LANE, _LANE)
    out = _emb_call(tok, tab)
    return out.reshape(_BATCH, _SEQ, _EMB)


# speedup vs baseline: 2.0291x; 2.0291x over previous
import math

import jax
import jax.numpy as jnp
from jax import lax
from jax.experimental import pallas as pl
from jax.experimental.pallas import tpu as pltpu
from jax.experimental.pallas import tpu_sc as plsc

_NC = 2
_NS = 16
_NW = _NC * _NS

_BATCH = 16384
_SEQ = 50
_EMB = 32
_TOTAL = _BATCH * _SEQ
_SCALE = math.sqrt(float(_EMB))

_LANE = 128
_TROW = 8
_CH = _TROW * _LANE
_ROWS_PER_W = _TOTAL // _NW // _LANE
_NCHUNK = _ROWS_PER_W // _TROW
_OUT_ROWS_CH = _CH * _EMB // _LANE


def _emb_body(tok_hbm, tab_hbm, out_hbm, idx_v, rows_v, out_v, sem):
    wid = lax.axis_index("s") * _NC + lax.axis_index("c")
    tok_base = wid * _ROWS_PER_W
    out_base = wid * (_ROWS_PER_W * _EMB)

    def chunk(c, carry):
        trow = tok_base + c * _TROW
        pltpu.sync_copy(tok_hbm.at[pl.ds(trow, _TROW)], idx_v)
        cps = [
            pltpu.async_copy(
                tab_hbm.at[idx_v.at[j]],
                rows_v.at[pl.ds(j * _LANE, _LANE)],
                sem,
            )
            for j in range(_TROW)
        ]
        for cp in cps:
            cp.wait()

        def scale(i, carry2):
            for u in range(4):
                for h in range(2):
                    out_v[i, pl.ds(u * 32 + h * 16, 16)] = (
                        rows_v[i * 4 + u, pl.ds(h * 16, 16)] * _SCALE
                    )
            return carry2

        lax.fori_loop(0, _OUT_ROWS_CH, scale, 0)
        pltpu.sync_copy(
            out_v, out_hbm.at[pl.ds(out_base + c * _OUT_ROWS_CH, _OUT_ROWS_CH)]
        )
        return carry

    lax.fori_loop(0, _NCHUNK, chunk, 0)


_mesh = plsc.VectorSubcoreMesh(
    core_axis_name="c", subcore_axis_name="s", num_cores=_NC, num_subcores=_NS
)

_emb_call = pl.kernel(
    _emb_body,
    out_type=jax.ShapeDtypeStruct((_TOTAL * _EMB // _LANE, _LANE), jnp.float32),
    mesh=_mesh,
    scratch_types=[
        pltpu.VMEM((_TROW, _LANE), jnp.int32),
        pltpu.VMEM((_CH, _EMB), jnp.float32),
        pltpu.VMEM((_OUT_ROWS_CH, _LANE), jnp.float32),
        pltpu.SemaphoreType.DMA,
    ],
    compiler_params=pltpu.CompilerParams(
        use_tc_tiling_on_sc=False, needs_layout_passes=False
    ),
)


@jax.jit
def kernel(tokens, embedding):
    tok = tokens.reshape(_TOTAL // _LANE, _LANE)
    out = _emb_call(tok, embedding)
    return out.reshape(_BATCH, _SEQ, _EMB)
